# baseline (speedup 1.0000x reference)
import jax
import jax.numpy as jnp
from jax import lax
from jax.experimental import pallas as pl
from jax.experimental.pallas import tpu as pltpu
from jax.experimental.pallas import tpu_sc as plsc

N = 10000
E = 320000
DIN = 128
H = 4
HD = 32
FEAT = H * HD
ROW = FEAT + 16

NC = 2
NS = 16
NW = NC * NS
EPW = E // NW
C = 40
NCHUNK = EPW // C
N_ACC = 10240
RPS = N_ACC // NS

M_BLK = 400
GRID1 = N // M_BLK


def _tc_pre_body(x_ref, wt_ref, b_ref, p_ref, hx_ref, s2_ref, m_ref, mx):
    i = pl.program_id(0)
    h = jnp.dot(x_ref[...], wt_ref[...], preferred_element_type=jnp.float32)
    h = h + b_ref[...]
    s = jnp.dot(h, p_ref[...], preferred_element_type=jnp.float32)
    hx_ref[:, 0:FEAT] = h
    hx_ref[:, FEAT:ROW] = s[:, 0:16]
    s2_ref[...] = s[:, 16:32]
    bm = jnp.max(s, axis=0, keepdims=True)

    @pl.when(i == 0)
    def _():
        mx[...] = bm

    @pl.when(i > 0)
    def _():
        mx[...] = jnp.maximum(mx[...], bm)

    t = mx[:, 0:16] + mx[:, 16:32]
    t = jnp.where(t > 0.0, t, 0.2 * t)
    lane = lax.broadcasted_iota(jnp.int32, (1, 16), 1)
    m_ref[...] = jnp.where(lane < H, t, 1e30)


def _tc_pre(x, wt, b, p):
    return pl.pallas_call(
        _tc_pre_body,
        grid=(GRID1,),
        in_specs=[
            pl.BlockSpec((M_BLK, DIN), lambda i: (i, 0)),
            pl.BlockSpec((DIN, FEAT), lambda i: (0, 0)),
            pl.BlockSpec((1, FEAT), lambda i: (0, 0)),
            pl.BlockSpec((DIN, 32), lambda i: (0, 0)),
        ],
        out_specs=[
            pl.BlockSpec((M_BLK, ROW), lambda i: (i, 0)),
            pl.BlockSpec((M_BLK, 16), lambda i: (i, 0)),
            pl.BlockSpec((1, 16), lambda i: (0, 0)),
        ],
        out_shape=[
            jax.ShapeDtypeStruct((N, ROW), jnp.float32),
            jax.ShapeDtypeStruct((N, 16), jnp.float32),
            jax.ShapeDtypeStruct((1, 16), jnp.float32),
        ],
        scratch_shapes=[pltpu.VMEM((1, 32), jnp.float32)],
    )(x, wt, b, p)


NBUF = 3
PRE = NCHUNK % NBUF + NBUF


def _sc_body(hx_hbm, s2_hbm, m_hbm, src_hbm, dst_hbm, out_hbm,
             srcs, dsts, hxr, s2r, orow, m_v, acc, gx, gs, ss):
    cid = lax.axis_index("c")
    sid = lax.axis_index("s")
    wid = sid * NC + cid

    zeros16 = jnp.zeros((16,), jnp.float32)

    @pl.loop(0, C)
    def _(r):
        for g in range(ROW // 16):
            orow[0][r, pl.ds(g * 16, 16)] = zeros16

    r0 = sid * RPS
    for t in range(RPS // C):
        pltpu.sync_copy(orow[0], acc.at[pl.ds(r0 + t * C, C)])
    plsc.subcore_barrier()

    pltpu.sync_copy(m_hbm, m_v)
    ebase = wid * EPW
    mvec = m_v[...]
    gdn = lax.GatherDimensionNumbers(
        offset_dims=(), collapsed_slice_dims=(0,), start_index_map=(0,))
    bidx = [jnp.full((16, 1), head, jnp.int32) for head in range(H)]

    def issue(k, b):
        base = ebase + k * C
        pltpu.sync_copy(src_hbm.at[pl.ds(base, C)], srcs[b])
        pltpu.sync_copy(dst_hbm.at[pl.ds(base, C)], dsts[b])
        pltpu.async_copy(hx_hbm.at[srcs[b]], hxr[b], gx[b])
        pltpu.async_copy(s2_hbm.at[dsts[b]], s2r[b], gs[b])

    def wait_gathers(b):
        pltpu.make_async_copy(hx_hbm.at[srcs[b]], hxr[b], gx[b]).wait()
        pltpu.make_async_copy(s2_hbm.at[dsts[b]], s2r[b], gs[b]).wait()

    def compute(b):
        @pl.loop(0, C, unroll=4)
        def _(c):
            e = hxr[b][c, pl.ds(FEAT, 16)] + s2r[b][c, pl.ds(0, 16)]
            e = jnp.where(e > 0.0, e, 0.2 * e)
            p = jnp.exp(e - mvec)
            orow[b][c, pl.ds(FEAT, 16)] = p
            for head in range(H):
                w = lax.gather(
                    p, bidx[head], dimension_numbers=gdn, slice_sizes=(1,),
                    mode=lax.GatherScatterMode.PROMISE_IN_BOUNDS)
                for g in range(HD // 16):
                    off = head * HD + g * 16
                    orow[b][c, pl.ds(off, 16)] = hxr[b][c, pl.ds(off, 16)] * w

    def start_scatter(k, b):
        pltpu.async_copy(orow[b], acc.at[dsts[b]], ss[b], add=True)

    def wait_scatter(b):
        pltpu.make_async_copy(orow[b], acc.at[dsts[0]], ss[b]).wait()

    for q in range(PRE):
        issue(q, 2)
        wait_gathers(2)
        compute(2)
        start_scatter(q, 2)
        if q < PRE - 1:
            wait_scatter(2)
    issue(PRE, 0)
    issue(PRE + 1, 1)

    @pl.loop(0, (NCHUNK - PRE) // NBUF)
    def _(t):
        for p in range(NBUF):
            k = PRE + t * NBUF + p
            brefill = (p + 2) % NBUF
            wait_gathers(p)
            compute(p)
            start_scatter(k, p)
            wait_scatter(brefill)
            krefill = k + 2

            @pl.when(krefill < NCHUNK)
            def _():
                issue(krefill, brefill)

    wait_scatter(2)
    plsc.subcore_barrier()
    pltpu.sync_copy(acc.at[pl.ds(r0, RPS)], out_hbm.at[cid, pl.ds(r0, RPS)])


def _sc_edge_pass(hx, s2, m, src, dst):
    mesh = plsc.VectorSubcoreMesh(core_axis_name="c", subcore_axis_name="s",
                                  num_cores=NC, num_subcores=NS)
    fn = pl.kernel(
        _sc_body,
        out_type=jax.ShapeDtypeStruct((NC, N_ACC, ROW), jnp.float32),
        mesh=mesh,
        scratch_types=[
            [pltpu.VMEM((C,), jnp.int32)] * NBUF,
            [pltpu.VMEM((C,), jnp.int32)] * NBUF,
            [pltpu.VMEM((C, ROW), jnp.float32)] * NBUF,
            [pltpu.VMEM((C, 16), jnp.float32)] * NBUF,
            [pltpu.VMEM((C, ROW), jnp.float32)] * NBUF,
            pltpu.VMEM((16,), jnp.float32),
            pltpu.VMEM_SHARED((N_ACC, ROW), jnp.float32),
            [pltpu.SemaphoreType.DMA] * NBUF,
            [pltpu.SemaphoreType.DMA] * NBUF,
            [pltpu.SemaphoreType.DMA] * NBUF,
        ],
        compiler_params=pltpu.CompilerParams(use_tc_tiling_on_sc=False,
                                             needs_layout_passes=False),
    )
    return fn(hx, s2, m, src, dst)


def _tc_post_body(part_ref, b_ref, out_ref):
    s = part_ref[0] + part_ref[1]
    num = s[:, 0:FEAT]
    den = jnp.dot(s[:, FEAT:ROW], b_ref[...],
                  preferred_element_type=jnp.float32)
    out_ref[...] = num / jnp.maximum(den, 1e-16)


def _tc_post(part, bmat):
    return pl.pallas_call(
        _tc_post_body,
        grid=(GRID1,),
        in_specs=[
            pl.BlockSpec((NC, M_BLK, ROW), lambda i: (0, i, 0)),
            pl.BlockSpec((16, FEAT), lambda i: (0, 0)),
        ],
        out_specs=pl.BlockSpec((M_BLK, FEAT), lambda i: (i, 0)),
        out_shape=jax.ShapeDtypeStruct((N, FEAT), jnp.float32),
    )(part, bmat)


def kernel(x, edge_index, W_w, W_b, attention):
    a1 = attention[:HD].reshape(HD, 1)
    a2 = attention[HD:].reshape(HD, 1)
    eye = jnp.eye(H, dtype=jnp.float32)
    p1 = jnp.pad(jnp.kron(eye, a1), ((0, 0), (0, 16 - H)))
    p2 = jnp.pad(jnp.kron(eye, a2), ((0, 0), (0, 16 - H)))
    pmat = jnp.concatenate([p1, p2], axis=1)
    bmat = jnp.pad(jnp.kron(eye, jnp.ones((1, HD), jnp.float32)),
                   ((0, 16 - H), (0, 0)))

    hx, s2, m = _tc_pre(x, W_w.T, W_b.reshape(1, FEAT), pmat)
    part = _sc_edge_pass(hx, s2, m.reshape(16), edge_index[0], edge_index[1])
    return _tc_post(part, bmat)


# speedup vs baseline: 2.7484x; 2.7484x over previous
import jax
import jax.numpy as jnp
from jax import lax
from jax.experimental import pallas as pl
from jax.experimental.pallas import tpu as pltpu
from jax.experimental.pallas import tpu_sc as plsc

N = 10000
E = 320000
DIN = 128
H = 4
HD = 32
FEAT = H * HD
ROW = FEAT + 16

NC = 2
NS = 16
NW = NC * NS
EPW = E // NW
C = 80
NCHUNK = EPW // C
N_ACC = 10240
RPS = N_ACC // NS

M_BLK = 400
GRID1 = N // M_BLK


def _tc_pre_body(x_ref, wt_ref, b_ref, p_ref, hx_ref, s2_ref, m_ref, mx):
    i = pl.program_id(0)
    h = jnp.dot(x_ref[...], wt_ref[...], preferred_element_type=jnp.float32)
    h = h + b_ref[...]
    s = jnp.dot(h, p_ref[...], preferred_element_type=jnp.float32)
    hx_ref[:, 0:FEAT] = h
    hx_ref[:, FEAT:ROW] = s[:, 0:16]
    s2_ref[...] = s[:, 16:32]
    bm = jnp.max(s, axis=0, keepdims=True)

    @pl.when(i == 0)
    def _():
        mx[...] = bm

    @pl.when(i > 0)
    def _():
        mx[...] = jnp.maximum(mx[...], bm)

    t = mx[:, 0:16] + mx[:, 16:32]
    t = jnp.where(t > 0.0, t, 0.2 * t)
    lane = lax.broadcasted_iota(jnp.int32, (1, 16), 1)
    m_ref[...] = jnp.where(lane < H, t, 1e30)


def _tc_pre(x, wt, b, p):
    return pl.pallas_call(
        _tc_pre_body,
        grid=(GRID1,),
        in_specs=[
            pl.BlockSpec((M_BLK, DIN), lambda i: (i, 0)),
            pl.BlockSpec((DIN, FEAT), lambda i: (0, 0)),
            pl.BlockSpec((1, FEAT), lambda i: (0, 0)),
            pl.BlockSpec((DIN, 32), lambda i: (0, 0)),
        ],
        out_specs=[
            pl.BlockSpec((M_BLK, ROW), lambda i: (i, 0)),
            pl.BlockSpec((M_BLK, 16), lambda i: (i, 0)),
            pl.BlockSpec((1, 16), lambda i: (0, 0)),
        ],
        out_shape=[
            jax.ShapeDtypeStruct((N, ROW), jnp.float32),
            jax.ShapeDtypeStruct((N, 16), jnp.float32),
            jax.ShapeDtypeStruct((1, 16), jnp.float32),
        ],
        scratch_shapes=[pltpu.VMEM((1, 32), jnp.float32)],
    )(x, wt, b, p)


NBUF = 3
PRE = NCHUNK % NBUF + NBUF


def _sc_body(hx_hbm, s2_hbm, m_hbm, src_hbm, dst_hbm, out_hbm,
             srcs, dsts, hxr, s2r, m_v, acc, gx, gs, ss):
    cid = lax.axis_index("c")
    sid = lax.axis_index("s")
    wid = sid * NC + cid

    zeros16 = jnp.zeros((16,), jnp.float32)

    @pl.loop(0, C)
    def _(r):
        for g in range(ROW // 16):
            hxr[0][r, pl.ds(g * 16, 16)] = zeros16

    r0 = sid * RPS
    for t in range(RPS // C):
        pltpu.sync_copy(hxr[0], acc.at[pl.ds(r0 + t * C, C)])
    plsc.subcore_barrier()

    pltpu.sync_copy(m_hbm, m_v)
    ebase = wid * EPW
    mvec = m_v[...]
    gdn = lax.GatherDimensionNumbers(
        offset_dims=(), collapsed_slice_dims=(0,), start_index_map=(0,))
    bidx = [jnp.full((16, 1), head, jnp.int32) for head in range(H)]

    def issue(k, b):
        base = ebase + k * C
        pltpu.sync_copy(src_hbm.at[pl.ds(base, C)], srcs[b])
        pltpu.sync_copy(dst_hbm.at[pl.ds(base, C)], dsts[b])
        pltpu.async_copy(hx_hbm.at[srcs[b]], hxr[b], gx[b])
        pltpu.async_copy(s2_hbm.at[dsts[b]], s2r[b], gs[b])

    def wait_gathers(b):
        pltpu.make_async_copy(hx_hbm.at[srcs[b]], hxr[b], gx[b]).wait()
        pltpu.make_async_copy(s2_hbm.at[dsts[b]], s2r[b], gs[b]).wait()

    def compute(b):
        @plsc.parallel_loop(0, C, unroll=4)
        def _(c):
            e = hxr[b][c, pl.ds(FEAT, 16)] + s2r[b][c, pl.ds(0, 16)]
            e = jnp.where(e > 0.0, e, 0.2 * e)
            p = jnp.exp(e - mvec)
            hxr[b][c, pl.ds(FEAT, 16)] = p
            for head in range(H):
                w = lax.gather(
                    p, bidx[head], dimension_numbers=gdn, slice_sizes=(1,),
                    mode=lax.GatherScatterMode.PROMISE_IN_BOUNDS)
                for g in range(HD // 16):
                    off = head * HD + g * 16
                    hxr[b][c, pl.ds(off, 16)] = hxr[b][c, pl.ds(off, 16)] * w

    def start_scatter(k, b):
        pltpu.async_copy(hxr[b], acc.at[dsts[b]], ss[b], add=True)

    def wait_scatter(b):
        pltpu.make_async_copy(hxr[b], acc.at[dsts[0]], ss[b]).wait()

    for q in range(PRE):
        issue(q, 2)
        wait_gathers(2)
        compute(2)
        start_scatter(q, 2)
        if q < PRE - 1:
            wait_scatter(2)
    issue(PRE, 0)
    issue(PRE + 1, 1)

    @pl.loop(0, (NCHUNK - PRE) // NBUF)
    def _(t):
        for p in range(NBUF):
            k = PRE + t * NBUF + p
            brefill = (p + 2) % NBUF
            wait_gathers(p)
            compute(p)
            start_scatter(k, p)
            wait_scatter(brefill)
            krefill = k + 2

            @pl.when(krefill < NCHUNK)
            def _():
                issue(krefill, brefill)

    wait_scatter(2)
    plsc.subcore_barrier()
    pltpu.sync_copy(acc.at[pl.ds(r0, RPS)], out_hbm.at[cid, pl.ds(r0, RPS)])


def _sc_edge_pass(hx, s2, m, src, dst):
    mesh = plsc.VectorSubcoreMesh(core_axis_name="c", subcore_axis_name="s",
                                  num_cores=NC, num_subcores=NS)
    fn = pl.kernel(
        _sc_body,
        out_type=jax.ShapeDtypeStruct((NC, N_ACC, ROW), jnp.float32),
        mesh=mesh,
        scratch_types=[
            [pltpu.VMEM((C,), jnp.int32)] * NBUF,
            [pltpu.VMEM((C,), jnp.int32)] * NBUF,
            [pltpu.VMEM((C, ROW), jnp.float32)] * NBUF,
            [pltpu.VMEM((C, 16), jnp.float32)] * NBUF,
            pltpu.VMEM((16,), jnp.float32),
            pltpu.VMEM_SHARED((N_ACC, ROW), jnp.float32),
            [pltpu.SemaphoreType.DMA] * NBUF,
            [pltpu.SemaphoreType.DMA] * NBUF,
            [pltpu.SemaphoreType.DMA] * NBUF,
        ],
        compiler_params=pltpu.CompilerParams(use_tc_tiling_on_sc=False,
                                             needs_layout_passes=False),
    )
    return fn(hx, s2, m, src, dst)


def _tc_post_body(part_ref, b_ref, out_ref):
    s = part_ref[0] + part_ref[1]
    num = s[:, 0:FEAT]
    den = jnp.dot(s[:, FEAT:ROW], b_ref[...],
                  preferred_element_type=jnp.float32)
    out_ref[...] = num / jnp.maximum(den, 1e-16)


def _tc_post(part, bmat):
    return pl.pallas_call(
        _tc_post_body,
        grid=(GRID1,),
        in_specs=[
            pl.BlockSpec((NC, M_BLK, ROW), lambda i: (0, i, 0)),
            pl.BlockSpec((16, FEAT), lambda i: (0, 0)),
        ],
        out_specs=pl.BlockSpec((M_BLK, FEAT), lambda i: (i, 0)),
        out_shape=jax.ShapeDtypeStruct((N, FEAT), jnp.float32),
    )(part, bmat)


def kernel(x, edge_index, W_w, W_b, attention):
    a1 = attention[:HD].reshape(HD, 1)
    a2 = attention[HD:].reshape(HD, 1)
    eye = jnp.eye(H, dtype=jnp.float32)
    p1 = jnp.pad(jnp.kron(eye, a1), ((0, 0), (0, 16 - H)))
    p2 = jnp.pad(jnp.kron(eye, a2), ((0, 0), (0, 16 - H)))
    pmat = jnp.concatenate([p1, p2], axis=1)
    bmat = jnp.pad(jnp.kron(eye, jnp.ones((1, HD), jnp.float32)),
                   ((0, 16 - H), (0, 0)))

    hx, s2, m = _tc_pre(x, W_w.T, W_b.reshape(1, FEAT), pmat)
    part = _sc_edge_pass(hx, s2, m.reshape(16), edge_index[0], edge_index[1])
    return _tc_post(part, bmat)


# speedup vs baseline: 2.9318x; 1.0667x over previous
import jax
import jax.numpy as jnp
from jax import lax
from jax.experimental import pallas as pl
from jax.experimental.pallas import tpu as pltpu
from jax.experimental.pallas import tpu_sc as plsc

N = 10000
E = 320000
DIN = 128
H = 4
HD = 32
FEAT = H * HD
ROW = FEAT + 16

NC = 2
NS = 16
NW = NC * NS
EPW = E // NW
C = 80
NCHUNK = EPW // C
N_ACC = 10240
RPS = N_ACC // NS

M_BLK = 400
GRID1 = N // M_BLK


def _tc_pre_body(x_ref, wt_ref, b_ref, p_ref, hx_ref, s2_ref, m_ref, mx):
    i = pl.program_id(0)
    h = jnp.dot(x_ref[...], wt_ref[...], preferred_element_type=jnp.float32)
    h = h + b_ref[...]
    s = jnp.dot(h, p_ref[...], preferred_element_type=jnp.float32)
    hx_ref[:, 0:FEAT] = h
    hx_ref[:, FEAT:ROW] = s[:, 0:16]
    s2_ref[...] = s[:, 16:32]
    bm = jnp.max(s, axis=0, keepdims=True)

    @pl.when(i == 0)
    def _():
        mx[...] = bm

    @pl.when(i > 0)
    def _():
        mx[...] = jnp.maximum(mx[...], bm)

    t = mx[:, 0:16] + mx[:, 16:32]
    t = jnp.where(t > 0.0, t, 0.2 * t)
    lane = lax.broadcasted_iota(jnp.int32, (1, 16), 1)
    m_ref[...] = jnp.where(lane < H, t, 1e30)


def _tc_pre(x, wt, b, p):
    return pl.pallas_call(
        _tc_pre_body,
        grid=(GRID1,),
        in_specs=[
            pl.BlockSpec((M_BLK, DIN), lambda i: (i, 0)),
            pl.BlockSpec((DIN, FEAT), lambda i: (0, 0)),
            pl.BlockSpec((1, FEAT), lambda i: (0, 0)),
            pl.BlockSpec((DIN, 32), lambda i: (0, 0)),
        ],
        out_specs=[
            pl.BlockSpec((M_BLK, ROW), lambda i: (i, 0)),
            pl.BlockSpec((M_BLK, 16), lambda i: (i, 0)),
            pl.BlockSpec((1, 16), lambda i: (0, 0)),
        ],
        out_shape=[
            jax.ShapeDtypeStruct((N, ROW), jnp.float32),
            jax.ShapeDtypeStruct((N, 16), jnp.float32),
            jax.ShapeDtypeStruct((1, 16), jnp.float32),
        ],
        scratch_shapes=[pltpu.VMEM((1, 32), jnp.float32)],
    )(x, wt, b, p)


NBUF = 3
PRE = NCHUNK % NBUF + NBUF


def _sc_body(hx_hbm, s2_hbm, m_hbm, ei_hbm, out_hbm,
             idx2, hxr, s2r, m_v, acc, gx, gs, ss):
    cid = lax.axis_index("c")
    sid = lax.axis_index("s")
    wid = sid * NC + cid

    zeros16 = jnp.zeros((16,), jnp.float32)

    @pl.loop(0, C)
    def _(r):
        for g in range(ROW // 16):
            hxr[0][r, pl.ds(g * 16, 16)] = zeros16

    r0 = sid * RPS
    for t in range(RPS // C):
        pltpu.sync_copy(hxr[0], acc.at[pl.ds(r0 + t * C, C)])
    plsc.subcore_barrier()

    pltpu.sync_copy(m_hbm, m_v)
    cbase = wid * NCHUNK
    mvec = m_v[...]
    gdn = lax.GatherDimensionNumbers(
        offset_dims=(), collapsed_slice_dims=(0,), start_index_map=(0,))
    bidx = [jnp.full((16, 1), head, jnp.int32) for head in range(H)]

    def issue(k, b):
        pltpu.sync_copy(ei_hbm.at[cbase + k], idx2[b])
        pltpu.async_copy(hx_hbm.at[idx2[b].at[0]], hxr[b], gx[b])
        pltpu.async_copy(s2_hbm.at[idx2[b].at[1]], s2r[b], gs[b])

    def wait_gathers(b):
        pltpu.make_async_copy(hx_hbm.at[idx2[b].at[0]], hxr[b], gx[b]).wait()
        pltpu.make_async_copy(s2_hbm.at[idx2[b].at[1]], s2r[b], gs[b]).wait()

    def compute(b):
        @plsc.parallel_loop(0, C, unroll=4)
        def _(c):
            e = hxr[b][c, pl.ds(FEAT, 16)] + s2r[b][c, pl.ds(0, 16)]
            e = jnp.where(e > 0.0, e, 0.2 * e)
            p = jnp.exp(e - mvec)
            hxr[b][c, pl.ds(FEAT, 16)] = p
            for head in range(H):
                w = lax.gather(
                    p, bidx[head], dimension_numbers=gdn, slice_sizes=(1,),
                    mode=lax.GatherScatterMode.PROMISE_IN_BOUNDS)
                for g in range(HD // 16):
                    off = head * HD + g * 16
                    hxr[b][c, pl.ds(off, 16)] = hxr[b][c, pl.ds(off, 16)] * w

    def start_scatter(k, b):
        pltpu.async_copy(hxr[b], acc.at[idx2[b].at[1]], ss[b], add=True)

    def wait_scatter(b):
        pltpu.make_async_copy(hxr[b], acc.at[idx2[b].at[1]], ss[b]).wait()

    for q in range(PRE):
        issue(q, 2)
        wait_gathers(2)
        compute(2)
        start_scatter(q, 2)
        if q < PRE - 1:
            wait_scatter(2)
    issue(PRE, 0)
    issue(PRE + 1, 1)

    @pl.loop(0, (NCHUNK - PRE) // NBUF)
    def _(t):
        for p in range(NBUF):
            k = PRE + t * NBUF + p
            brefill = (p + 2) % NBUF
            wait_gathers(p)
            compute(p)
            start_scatter(k, p)
            wait_scatter(brefill)
            krefill = k + 2

            @pl.when(krefill < NCHUNK)
            def _():
                issue(krefill, brefill)

    wait_scatter(2)
    plsc.subcore_barrier()
    pltpu.sync_copy(acc.at[pl.ds(r0, RPS)], out_hbm.at[cid, pl.ds(r0, RPS)])


def _sc_edge_pass(hx, s2, m, ei3):
    mesh = plsc.VectorSubcoreMesh(core_axis_name="c", subcore_axis_name="s",
                                  num_cores=NC, num_subcores=NS)
    fn = pl.kernel(
        _sc_body,
        out_type=jax.ShapeDtypeStruct((NC, N_ACC, ROW), jnp.float32),
        mesh=mesh,
        scratch_types=[
            [pltpu.VMEM((2, C), jnp.int32)] * NBUF,
            [pltpu.VMEM((C, ROW), jnp.float32)] * NBUF,
            [pltpu.VMEM((C, 16), jnp.float32)] * NBUF,
            pltpu.VMEM((16,), jnp.float32),
            pltpu.VMEM_SHARED((N_ACC, ROW), jnp.float32),
            [pltpu.SemaphoreType.DMA] * NBUF,
            [pltpu.SemaphoreType.DMA] * NBUF,
            [pltpu.SemaphoreType.DMA] * NBUF,
        ],
        compiler_params=pltpu.CompilerParams(use_tc_tiling_on_sc=False,
                                             needs_layout_passes=False),
    )
    return fn(hx, s2, m, ei3)


def _tc_post_body(part_ref, b_ref, out_ref):
    s = part_ref[0] + part_ref[1]
    num = s[:, 0:FEAT]
    den = jnp.dot(s[:, FEAT:ROW], b_ref[...],
                  preferred_element_type=jnp.float32)
    out_ref[...] = num / jnp.maximum(den, 1e-16)


def _tc_post(part, bmat):
    return pl.pallas_call(
        _tc_post_body,
        grid=(GRID1,),
        in_specs=[
            pl.BlockSpec((NC, M_BLK, ROW), lambda i: (0, i, 0)),
            pl.BlockSpec((16, FEAT), lambda i: (0, 0)),
        ],
        out_specs=pl.BlockSpec((M_BLK, FEAT), lambda i: (i, 0)),
        out_shape=jax.ShapeDtypeStruct((N, FEAT), jnp.float32),
    )(part, bmat)


def kernel(x, edge_index, W_w, W_b, attention):
    a1 = attention[:HD].reshape(HD, 1)
    a2 = attention[HD:].reshape(HD, 1)
    eye = jnp.eye(H, dtype=jnp.float32)
    p1 = jnp.pad(jnp.kron(eye, a1), ((0, 0), (0, 16 - H)))
    p2 = jnp.pad(jnp.kron(eye, a2), ((0, 0), (0, 16 - H)))
    pmat = jnp.concatenate([p1, p2], axis=1)
    bmat = jnp.pad(jnp.kron(eye, jnp.ones((1, HD), jnp.float32)),
                   ((0, 16 - H), (0, 0)))

    hx, s2, m = _tc_pre(x, W_w.T, W_b.reshape(1, FEAT), pmat)
    ei3 = jnp.stack([edge_index[0].reshape(E // C, C),
                     edge_index[1].reshape(E // C, C)], axis=1)
    part = _sc_edge_pass(hx, s2, m.reshape(16), ei3)
    return _tc_post(part, bmat)


# speedup vs baseline: 2.9537x; 1.0075x over previous
import jax
import jax.numpy as jnp
from jax import lax
from jax.experimental import pallas as pl
from jax.experimental.pallas import tpu as pltpu
from jax.experimental.pallas import tpu_sc as plsc

N = 10000
E = 320000
DIN = 128
H = 4
HD = 32
FEAT = H * HD
ROW = FEAT + 16

NC = 2
NS = 16
NW = NC * NS
EPW = E // NW
C = 80
NCHUNK = EPW // C
N_ACC = 10240
RPS = N_ACC // NS

M_BLK = 400
GRID1 = N // M_BLK


def _tc_pre_body(x_ref, wt_ref, b_ref, p_ref, hx_ref, s2_ref, m_ref, mx):
    i = pl.program_id(0)
    h = lax.dot_general(x_ref[...], wt_ref[...], (((1,), (1,)), ((), ())),
                        preferred_element_type=jnp.float32)
    h = h + b_ref[...]
    s = jnp.dot(h, p_ref[...], preferred_element_type=jnp.float32)
    hx_ref[:, 0:FEAT] = h
    hx_ref[:, FEAT:ROW] = s[:, 0:16]
    s2_ref[...] = s[:, 16:32]
    bm = jnp.max(s, axis=0, keepdims=True)

    @pl.when(i == 0)
    def _():
        mx[...] = bm

    @pl.when(i > 0)
    def _():
        mx[...] = jnp.maximum(mx[...], bm)

    t = mx[:, 0:16] + mx[:, 16:32]
    t = jnp.where(t > 0.0, t, 0.2 * t)
    lane = lax.broadcasted_iota(jnp.int32, (1, 16), 1)
    m_ref[...] = jnp.where(lane < H, t, 1e30)


def _tc_pre(x, wt, b, p):
    return pl.pallas_call(
        _tc_pre_body,
        grid=(GRID1,),
        in_specs=[
            pl.BlockSpec((M_BLK, DIN), lambda i: (i, 0)),
            pl.BlockSpec((DIN, FEAT), lambda i: (0, 0)),
            pl.BlockSpec((1, FEAT), lambda i: (0, 0)),
            pl.BlockSpec((DIN, 32), lambda i: (0, 0)),
        ],
        out_specs=[
            pl.BlockSpec((M_BLK, ROW), lambda i: (i, 0)),
            pl.BlockSpec((M_BLK, 16), lambda i: (i, 0)),
            pl.BlockSpec((1, 16), lambda i: (0, 0)),
        ],
        out_shape=[
            jax.ShapeDtypeStruct((N, ROW), jnp.float32),
            jax.ShapeDtypeStruct((N, 16), jnp.float32),
            jax.ShapeDtypeStruct((1, 16), jnp.float32),
        ],
        scratch_shapes=[pltpu.VMEM((1, 32), jnp.float32)],
    )(x, wt, b, p)


NBUF = 3
PRE = NCHUNK % NBUF + NBUF


def _sc_body(hx_hbm, s2_hbm, m_hbm, ei_hbm, out_hbm,
             idx2, hxr, s2r, m_v, acc, gx, gs, ss):
    cid = lax.axis_index("c")
    sid = lax.axis_index("s")
    wid = sid * NC + cid

    zeros16 = jnp.zeros((16,), jnp.float32)

    @pl.loop(0, C)
    def _(r):
        for g in range(ROW // 16):
            hxr[0][r, pl.ds(g * 16, 16)] = zeros16

    r0 = sid * RPS
    for t in range(RPS // C):
        pltpu.sync_copy(hxr[0], acc.at[pl.ds(r0 + t * C, C)])
    plsc.subcore_barrier()

    pltpu.sync_copy(m_hbm, m_v)
    cbase = wid * NCHUNK
    mvec = m_v[...]
    gdn = lax.GatherDimensionNumbers(
        offset_dims=(), collapsed_slice_dims=(0,), start_index_map=(0,))
    bidx = [jnp.full((16, 1), head, jnp.int32) for head in range(H)]

    def issue(k, b):
        pltpu.sync_copy(ei_hbm.at[cbase + k], idx2[b])
        pltpu.async_copy(hx_hbm.at[idx2[b].at[0]], hxr[b], gx[b])
        pltpu.async_copy(s2_hbm.at[idx2[b].at[1]], s2r[b], gs[b])

    def wait_gathers(b):
        pltpu.make_async_copy(hx_hbm.at[idx2[b].at[0]], hxr[b], gx[b]).wait()
        pltpu.make_async_copy(s2_hbm.at[idx2[b].at[1]], s2r[b], gs[b]).wait()

    def compute(b):
        @plsc.parallel_loop(0, C, unroll=4)
        def _(c):
            e = hxr[b][c, pl.ds(FEAT, 16)] + s2r[b][c, pl.ds(0, 16)]
            e = jnp.where(e > 0.0, e, 0.2 * e)
            p = jnp.exp(e - mvec)
            hxr[b][c, pl.ds(FEAT, 16)] = p
            for head in range(H):
                w = lax.gather(
                    p, bidx[head], dimension_numbers=gdn, slice_sizes=(1,),
                    mode=lax.GatherScatterMode.PROMISE_IN_BOUNDS)
                for g in range(HD // 16):
                    off = head * HD + g * 16
                    hxr[b][c, pl.ds(off, 16)] = hxr[b][c, pl.ds(off, 16)] * w

    def start_scatter(k, b):
        pltpu.async_copy(hxr[b], acc.at[idx2[b].at[1]], ss[b], add=True)

    def wait_scatter(b):
        pltpu.make_async_copy(hxr[b], acc.at[idx2[b].at[1]], ss[b]).wait()

    for q in range(PRE):
        issue(q, 2)
        wait_gathers(2)
        compute(2)
        start_scatter(q, 2)
        if q < PRE - 1:
            wait_scatter(2)
    issue(PRE, 0)
    issue(PRE + 1, 1)

    @pl.loop(0, (NCHUNK - PRE) // NBUF)
    def _(t):
        for p in range(NBUF):
            k = PRE + t * NBUF + p
            brefill = (p + 2) % NBUF
            wait_gathers(p)
            compute(p)
            start_scatter(k, p)
            wait_scatter(brefill)
            krefill = k + 2

            @pl.when(krefill < NCHUNK)
            def _():
                issue(krefill, brefill)

    wait_scatter(2)
    plsc.subcore_barrier()
    pltpu.sync_copy(acc.at[pl.ds(r0, RPS)], out_hbm.at[cid, pl.ds(r0, RPS)])


def _sc_edge_pass(hx, s2, m, ei3):
    mesh = plsc.VectorSubcoreMesh(core_axis_name="c", subcore_axis_name="s",
                                  num_cores=NC, num_subcores=NS)
    fn = pl.kernel(
        _sc_body,
        out_type=jax.ShapeDtypeStruct((NC, N_ACC, ROW), jnp.float32),
        mesh=mesh,
        scratch_types=[
            [pltpu.VMEM((2, C), jnp.int32)] * NBUF,
            [pltpu.VMEM((C, ROW), jnp.float32)] * NBUF,
            [pltpu.VMEM((C, 16), jnp.float32)] * NBUF,
            pltpu.VMEM((16,), jnp.float32),
            pltpu.VMEM_SHARED((N_ACC, ROW), jnp.float32),
            [pltpu.SemaphoreType.DMA] * NBUF,
            [pltpu.SemaphoreType.DMA] * NBUF,
            [pltpu.SemaphoreType.DMA] * NBUF,
        ],
        compiler_params=pltpu.CompilerParams(use_tc_tiling_on_sc=False,
                                             needs_layout_passes=False),
    )
    return fn(hx, s2, m, ei3)


def _tc_post_body(part_ref, b_ref, out_ref):
    s = part_ref[0] + part_ref[1]
    num = s[:, 0:FEAT]
    den = jnp.dot(s[:, FEAT:ROW], b_ref[...],
                  preferred_element_type=jnp.float32)
    out_ref[...] = num / jnp.maximum(den, 1e-16)


def _tc_post(part, bmat):
    return pl.pallas_call(
        _tc_post_body,
        grid=(GRID1,),
        in_specs=[
            pl.BlockSpec((NC, M_BLK, ROW), lambda i: (0, i, 0)),
            pl.BlockSpec((16, FEAT), lambda i: (0, 0)),
        ],
        out_specs=pl.BlockSpec((M_BLK, FEAT), lambda i: (i, 0)),
        out_shape=jax.ShapeDtypeStruct((N, FEAT), jnp.float32),
    )(part, bmat)


def kernel(x, edge_index, W_w, W_b, attention):
    a1 = attention[:HD].reshape(HD, 1)
    a2 = attention[HD:].reshape(HD, 1)
    eye = jnp.eye(H, dtype=jnp.float32)
    p1 = jnp.pad(jnp.kron(eye, a1), ((0, 0), (0, 16 - H)))
    p2 = jnp.pad(jnp.kron(eye, a2), ((0, 0), (0, 16 - H)))
    pmat = jnp.concatenate([p1, p2], axis=1)
    bmat = jnp.pad(jnp.kron(eye, jnp.ones((1, HD), jnp.float32)),
                   ((0, 16 - H), (0, 0)))

    hx, s2, m = _tc_pre(x, W_w, W_b.reshape(1, FEAT), pmat)
    ei3 = jnp.stack([edge_index[0].reshape(E // C, C),
                     edge_index[1].reshape(E // C, C)], axis=1)
    part = _sc_edge_pass(hx, s2, m.reshape(16), ei3)
    return _tc_post(part, bmat)
